# baseline (speedup 1.0000x reference)
import functools

import jax
import jax.numpy as jnp
from jax import lax
from jax.experimental import pallas as pl
from jax.experimental.pallas import tpu as pltpu
from jax.experimental.pallas import tpu_sc as plsc

RO = 4.0
CH = 64
H = 128
W = 128
NC_CONT = 64
P = 128
NPTS = NC_CONT * P
NWORK = 32
PTS_W = NPTS // NWORK
NB = 16


def _sc_gather(feat_rows, cur, base, scale):
    mesh = plsc.VectorSubcoreMesh(
        core_axis_name="c", subcore_axis_name="s", num_cores=2, num_subcores=16
    )
    pts_w = NPTS // NWORK
    out_t = tuple(
        jax.ShapeDtypeStruct((NPTS, 2 * CH), jnp.float32) for _ in range(2)
    )
    scratch = [
        pltpu.VMEM((pts_w, 2), jnp.float32),
        pltpu.VMEM((pts_w,), jnp.int32),
        pltpu.VMEM((4, 2, 128), jnp.int32),
        pltpu.VMEM((4, pts_w, CH), jnp.float32),
        pltpu.SemaphoreType.DMA,
    ]

    @functools.partial(pl.kernel, out_type=out_t, mesh=mesh,
                       scratch_types=scratch,
                       compiler_params=pltpu.CompilerParams(
                           use_tc_tiling_on_sc=False,
                           needs_layout_passes=False))
    def k(feat_h, cur_h, b_h, o0, o1, xyv, bv, idxv, rows, sem):
        wid = lax.axis_index("s") * 2 + lax.axis_index("c")
        pbase = wid * pts_w
        pltpu.sync_copy(cur_h.at[pl.ds(pbase, pts_w)], xyv)
        pltpu.sync_copy(b_h.at[pl.ds(pbase, pts_w)], bv)

        col0 = jnp.zeros((16,), jnp.int32)
        col1 = jnp.ones((16,), jnp.int32)
        lane = lax.iota(jnp.int32, 16)
        for i in range(pts_w // 16):
            pp = lane + (i * 16)
            x = plsc.load_gather(xyv, [pp, col0])
            y = plsc.load_gather(xyv, [pp, col1])
            b = bv[pl.ds(i * 16, 16)]
            if scale != 1.0:
                x = x * scale
                y = y * scale
            xc = jnp.minimum(jnp.maximum(x, 0.0), float(W - 1))
            yc = jnp.minimum(jnp.maximum(y, 0.0), float(H - 1))
            x0 = xc.astype(jnp.int32)
            y0 = yc.astype(jnp.int32)
            x1 = jnp.minimum(x0 + 1, W - 1)
            y1 = jnp.minimum(y0 + 1, H - 1)
            row0 = b + y0 * W
            row1 = b + y1 * W
            half = i // 8
            off = pl.ds((i % 8) * 16, 16)
            idxv[0, half, off] = row0 + x0
            idxv[1, half, off] = row0 + x1
            idxv[2, half, off] = row1 + x0
            idxv[3, half, off] = row1 + x1

        copies = []
        for n in range(4):
            for h in range(2):
                copies.append(pltpu.async_copy(
                    feat_h.at[idxv.at[n, h]],
                    rows.at[n, pl.ds(h * 128, 128)],
                    sem,
                ))
        for c in copies:
            c.wait()

        for n, (out, cs) in enumerate((
                (o0, 0), (o0, CH), (o1, 0), (o1, CH))):
            pltpu.sync_copy(
                rows.at[n],
                out.at[pl.ds(pbase, pts_w), pl.ds(cs, CH)])

    return k(feat_rows, cur, base)


YC = 2
NPAIR = (H // 2) // (2 * YC)
CROWS = CH * YC
OPIX = YC * W


def _sc_nhwc(cnn2):
    mesh = plsc.VectorSubcoreMesh(
        core_axis_name="c", subcore_axis_name="s", num_cores=2, num_subcores=16
    )
    n_img = cnn2.shape[0] // (CH * H)
    out_t = jax.ShapeDtypeStruct((n_img * H * W, CH), jnp.float32)
    scratch = [
        pltpu.VMEM((2, 128), jnp.int32),
        pltpu.VMEM((2, CROWS, W), jnp.float32),
        pltpu.VMEM((2, OPIX, CH), jnp.float32),
        pltpu.SemaphoreType.DMA,
        pltpu.SemaphoreType.DMA,
    ]

    @functools.partial(pl.kernel, out_type=out_t, mesh=mesh,
                       scratch_types=scratch,
                       compiler_params=pltpu.CompilerParams(
                           use_tc_tiling_on_sc=False,
                           needs_layout_passes=False))
    def k(cnn_h, out_h, idxv, inb, outb, sem_in, sem_out):
        wid = lax.axis_index("s") * 2 + lax.axis_index("c")
        n = wid // 2
        y_base = (wid % 2) * (H // 2)
        lane = lax.iota(jnp.int32, 16)

        def fire_gather(t, buf):
            y0 = y_base + t * YC
            for kk in range(128 // 16):
                j = lane + kk * 16
                idxv[buf, pl.ds(kk * 16, 16)] = (
                    (n * CH + (j & 63)) * H + y0 + (j >> 6))
            return pltpu.async_copy(
                cnn_h.at[idxv.at[buf]], inb.at[buf], sem_in)

        def transpose_chunk(buf):
            @plsc.parallel_loop(0, CROWS, unroll=8)
            def body(r):
                yi = r >> 6
                c = r & 63
                rbase = yi * W
                for xc in range(W // 16):
                    x = lane + xc * 16
                    v = inb[buf, r, pl.ds(xc * 16, 16)]
                    plsc.store_scatter(
                        outb.at[buf],
                        [rbase + x, jnp.broadcast_to(c, (16,))], v)

        def out_copy(t, buf):
            pix0 = (n * H + y_base + t * YC) * W
            return pltpu.async_copy(
                outb.at[buf], out_h.at[pl.ds(pix0, OPIX)], sem_out)

        def pair(q, _):
            t0 = q * 2
            g0 = fire_gather(t0, 0)
            g1 = fire_gather(t0 + 1, 1)
            g0.wait()
            transpose_chunk(0)
            o0 = out_copy(t0, 0)
            g1.wait()
            transpose_chunk(1)
            o1 = out_copy(t0 + 1, 1)
            o0.wait()
            o1.wait()
            return 0

        lax.fori_loop(0, NPAIR, pair, 0)

    return k(cnn2)


def _avg9(x):
    s = x + jnp.roll(x, 1, axis=1)
    s = s + jnp.roll(s, 2, axis=1)
    s = s + jnp.roll(s, 4, axis=1)
    s = s + jnp.roll(x, 8, axis=1)
    return jnp.roll(s, -4, axis=1) * (1.0 / 9.0)


def _tc_resgcn_body(scale, last, g0r, g1r, itr,
                    W1fr, W1cr, b1r, W2tr, b2r, Whtr, bhr, outr):
    it = itr[...]
    if scale != 1.0:
        it = it * scale
    xy = jnp.clip(it, 0.0, float(W - 1))
    wxy = xy - jnp.floor(xy)
    wx = wxy[:, :, 0:1]
    wy = wxy[:, :, 1:2]
    g0 = g0r[...]
    g1 = g1r[...]
    feat = (g0[:, :, :CH] * (1.0 - wx) * (1.0 - wy)
            + g0[:, :, CH:] * wx * (1.0 - wy)
            + g1[:, :, :CH] * (1.0 - wx) * wy
            + g1[:, :, CH:] * wx * wy)
    c = (it - jnp.min(it, axis=1, keepdims=True)) * RO

    DH = W2tr.shape[0]
    bf = jnp.bfloat16
    Af = _avg9(feat).reshape(NB * P, CH)
    Ac = _avg9(c).reshape(NB * P, 2)
    h1 = jnp.dot(Af.astype(bf), W1fr[...].astype(bf),
                 preferred_element_type=jnp.float32)
    h1 = h1 + jnp.dot(Ac, W1cr[...], preferred_element_type=jnp.float32)
    h1 = jnp.maximum(h1 + b1r[...], 0.0)
    Ah = _avg9(h1.reshape(NB, P, DH)).reshape(NB * P, DH)
    h2 = jnp.dot(Ah.astype(bf), W2tr[...].astype(bf),
                 preferred_element_type=jnp.float32)
    h2 = jnp.maximum(h2 + b2r[...], 0.0)
    hh = h1 + h2
    off = (jnp.dot(hh.astype(bf), Whtr[...].astype(bf),
                   preferred_element_type=jnp.float32)
           + bhr[...]).reshape(NB, P, 2)
    if last:
        outr[...] = it * RO + off
    else:
        outr[...] = it + off * (1.0 / RO)


def _tc_resgcn(g0, g1, i_it, W1f, W1c, b1, W2t, b2, Wht, bh,
               scale, last, nc=NC_CONT):
    DH = W2t.shape[0]
    fspec = pl.BlockSpec((NB, P, 2 * CH), lambda i: (i, 0, 0))
    ispec = pl.BlockSpec((NB, P, 2), lambda i: (i, 0, 0))
    wspec2 = lambda a, b: pl.BlockSpec((a, b), lambda i: (0, 0))
    return pl.pallas_call(
        functools.partial(_tc_resgcn_body, scale, last),
        grid=(nc // NB,),
        in_specs=[
            fspec, fspec, ispec,
            wspec2(CH, DH),
            wspec2(2, DH),
            wspec2(1, DH),
            wspec2(DH, DH),
            wspec2(1, DH),
            wspec2(DH, 2),
            wspec2(1, 2),
        ],
        out_specs=ispec,
        out_shape=jax.ShapeDtypeStruct((nc, P, 2), jnp.float32),
        compiler_params=pltpu.CompilerParams(
            dimension_semantics=("arbitrary",),
        ),
    )(g0, g1, i_it, W1f, W1c, b1, W2t, b2, Wht, bh)


def kernel(cnn_feature, i_it_py, py_ind,
           W1_0, b1_0, W2_0, b2_0, Wh_0, bh_0,
           W1_1, b1_1, W2_1, b2_1, Wh_1, bh_1,
           W1_2, b1_2, W2_2, b2_2, Wh_2, bh_2):
    feat_rows = _sc_nhwc(cnn_feature.reshape(-1, W))
    base = jnp.broadcast_to(
        (py_ind.astype(jnp.int32) * (H * W))[:, None], (NC_CONT, P)
    ).reshape(NPTS)

    params = []
    for (W1, b1, W2, b2, Wh, bh) in (
        (W1_0, b1_0, W2_0, b2_0, Wh_0, bh_0),
        (W1_1, b1_1, W2_1, b2_1, Wh_1, bh_1),
        (W1_2, b1_2, W2_2, b2_2, Wh_2, bh_2),
    ):
        params.append((
            W1[:, :CH].T,
            W1[:, CH:].T,
            b1[None, :],
            W2.T,
            b2[None, :],
            Wh.T,
            bh[None, :],
        ))

    cur = i_it_py
    scale = float(W - 1)
    for m in range(3):
        g0, g1 = _sc_gather(feat_rows, cur.reshape(NPTS, 2), base, scale)
        cc = 2 * CH
        cur = _tc_resgcn(
            g0.reshape(NC_CONT, P, cc), g1.reshape(NC_CONT, P, cc),
            cur, *params[m], scale=scale, last=(m == 2),
        )
        scale = 1.0
    return cur


# speedup vs baseline: 1.3586x; 1.3586x over previous
import functools

import jax
import jax.numpy as jnp
from jax import lax
from jax.experimental import pallas as pl
from jax.experimental.pallas import tpu as pltpu
from jax.experimental.pallas import tpu_sc as plsc

RO = 4.0
CH = 64
H = 128
W = 128
NC_CONT = 64
P = 128
NPTS = NC_CONT * P
NWORK = 32
PTS_W = NPTS // NWORK
NB = 16


def _sc_gather(feat_rows, cur, base, scale):
    mesh = plsc.VectorSubcoreMesh(
        core_axis_name="c", subcore_axis_name="s", num_cores=2, num_subcores=16
    )
    pts_w = NPTS // NWORK
    out_t = tuple(
        jax.ShapeDtypeStruct((NPTS, 2 * CH), jnp.float32) for _ in range(2)
    )
    scratch = [
        pltpu.VMEM((pts_w, 2), jnp.float32),
        pltpu.VMEM((pts_w,), jnp.int32),
        pltpu.VMEM((4, 2, 128), jnp.int32),
        pltpu.VMEM((4, pts_w, CH), jnp.float32),
        pltpu.SemaphoreType.DMA,
    ]

    @functools.partial(pl.kernel, out_type=out_t, mesh=mesh,
                       scratch_types=scratch,
                       compiler_params=pltpu.CompilerParams(
                           use_tc_tiling_on_sc=False,
                           needs_layout_passes=False))
    def k(feat_h, cur_h, b_h, o0, o1, xyv, bv, idxv, rows, sem):
        wid = lax.axis_index("s") * 2 + lax.axis_index("c")
        pbase = wid * pts_w
        pltpu.sync_copy(cur_h.at[pl.ds(pbase, pts_w)], xyv)
        pltpu.sync_copy(b_h.at[pl.ds(pbase, pts_w)], bv)

        col0 = jnp.zeros((16,), jnp.int32)
        col1 = jnp.ones((16,), jnp.int32)
        lane = lax.iota(jnp.int32, 16)
        for i in range(pts_w // 16):
            pp = lane + (i * 16)
            x = plsc.load_gather(xyv, [pp, col0])
            y = plsc.load_gather(xyv, [pp, col1])
            b = bv[pl.ds(i * 16, 16)]
            if scale != 1.0:
                x = x * scale
                y = y * scale
            xc = jnp.minimum(jnp.maximum(x, 0.0), float(W - 1))
            yc = jnp.minimum(jnp.maximum(y, 0.0), float(H - 1))
            x0 = xc.astype(jnp.int32)
            y0 = yc.astype(jnp.int32)
            x1 = jnp.minimum(x0 + 1, W - 1)
            y1 = jnp.minimum(y0 + 1, H - 1)
            row0 = b + y0 * W
            row1 = b + y1 * W
            half = i // 8
            off = pl.ds((i % 8) * 16, 16)
            idxv[0, half, off] = row0 + x0
            idxv[1, half, off] = row0 + x1
            idxv[2, half, off] = row1 + x0
            idxv[3, half, off] = row1 + x1

        copies = []
        for n in range(4):
            for h in range(2):
                copies.append(pltpu.async_copy(
                    feat_h.at[idxv.at[n, h]],
                    rows.at[n, pl.ds(h * 128, 128)],
                    sem,
                ))
        for c in copies:
            c.wait()

        for n, (out, cs) in enumerate((
                (o0, 0), (o0, CH), (o1, 0), (o1, CH))):
            pltpu.sync_copy(
                rows.at[n],
                out.at[pl.ds(pbase, pts_w), pl.ds(cs, CH)])

    return k(feat_rows, cur, base)


def _avg9(x):
    s = x + jnp.roll(x, 1, axis=1)
    s = s + jnp.roll(s, 2, axis=1)
    s = s + jnp.roll(s, 4, axis=1)
    s = s + jnp.roll(x, 8, axis=1)
    return jnp.roll(s, -4, axis=1) * (1.0 / 9.0)


def _tc_resgcn_body(scale, last, g0r, g1r, itr,
                    W1fr, W1cr, b1r, W2tr, b2r, Whtr, bhr, outr):
    it = itr[...]
    if scale != 1.0:
        it = it * scale
    xy = jnp.clip(it, 0.0, float(W - 1))
    wxy = xy - jnp.floor(xy)
    wx = wxy[:, :, 0:1]
    wy = wxy[:, :, 1:2]
    g0 = g0r[...]
    g1 = g1r[...]
    feat = (g0[:, :, :CH] * (1.0 - wx) * (1.0 - wy)
            + g0[:, :, CH:] * wx * (1.0 - wy)
            + g1[:, :, :CH] * (1.0 - wx) * wy
            + g1[:, :, CH:] * wx * wy)
    c = (it - jnp.min(it, axis=1, keepdims=True)) * RO

    DH = W2tr.shape[0]
    bf = jnp.bfloat16
    Af = _avg9(feat).reshape(NB * P, CH)
    Ac = _avg9(c).reshape(NB * P, 2)
    h1 = jnp.dot(Af.astype(bf), W1fr[...].astype(bf),
                 preferred_element_type=jnp.float32)
    h1 = h1 + jnp.dot(Ac, W1cr[...], preferred_element_type=jnp.float32)
    h1 = jnp.maximum(h1 + b1r[...], 0.0)
    Ah = _avg9(h1.reshape(NB, P, DH)).reshape(NB * P, DH)
    h2 = jnp.dot(Ah.astype(bf), W2tr[...].astype(bf),
                 preferred_element_type=jnp.float32)
    h2 = jnp.maximum(h2 + b2r[...], 0.0)
    hh = h1 + h2
    off = (jnp.dot(hh.astype(bf), Whtr[...].astype(bf),
                   preferred_element_type=jnp.float32)
           + bhr[...]).reshape(NB, P, 2)
    if last:
        outr[...] = it * RO + off
    else:
        outr[...] = it + off * (1.0 / RO)


def _tc_resgcn(g0, g1, i_it, W1f, W1c, b1, W2t, b2, Wht, bh,
               scale, last, nc=NC_CONT):
    DH = W2t.shape[0]
    fspec = pl.BlockSpec((NB, P, 2 * CH), lambda i: (i, 0, 0))
    ispec = pl.BlockSpec((NB, P, 2), lambda i: (i, 0, 0))
    wspec2 = lambda a, b: pl.BlockSpec((a, b), lambda i: (0, 0))
    return pl.pallas_call(
        functools.partial(_tc_resgcn_body, scale, last),
        grid=(nc // NB,),
        in_specs=[
            fspec, fspec, ispec,
            wspec2(CH, DH),
            wspec2(2, DH),
            wspec2(1, DH),
            wspec2(DH, DH),
            wspec2(1, DH),
            wspec2(DH, 2),
            wspec2(1, 2),
        ],
        out_specs=ispec,
        out_shape=jax.ShapeDtypeStruct((nc, P, 2), jnp.float32),
        compiler_params=pltpu.CompilerParams(
            dimension_semantics=("arbitrary",),
        ),
    )(g0, g1, i_it, W1f, W1c, b1, W2t, b2, Wht, bh)


def kernel(cnn_feature, i_it_py, py_ind,
           W1_0, b1_0, W2_0, b2_0, Wh_0, bh_0,
           W1_1, b1_1, W2_1, b2_1, Wh_1, bh_1,
           W1_2, b1_2, W2_2, b2_2, Wh_2, bh_2):
    feat_rows = cnn_feature.transpose(0, 2, 3, 1).reshape(-1, CH)
    base = jnp.broadcast_to(
        (py_ind.astype(jnp.int32) * (H * W))[:, None], (NC_CONT, P)
    ).reshape(NPTS)

    params = []
    for (W1, b1, W2, b2, Wh, bh) in (
        (W1_0, b1_0, W2_0, b2_0, Wh_0, bh_0),
        (W1_1, b1_1, W2_1, b2_1, Wh_1, bh_1),
        (W1_2, b1_2, W2_2, b2_2, Wh_2, bh_2),
    ):
        params.append((
            W1[:, :CH].T,
            W1[:, CH:].T,
            b1[None, :],
            W2.T,
            b2[None, :],
            Wh.T,
            bh[None, :],
        ))

    cur = i_it_py
    scale = float(W - 1)
    for m in range(3):
        g0, g1 = _sc_gather(feat_rows, cur.reshape(NPTS, 2), base, scale)
        cc = 2 * CH
        cur = _tc_resgcn(
            g0.reshape(NC_CONT, P, cc), g1.reshape(NC_CONT, P, cc),
            cur, *params[m], scale=scale, last=(m == 2),
        )
        scale = 1.0
    return cur
